# baseline (speedup 1.0000x reference)
import functools
import jax
import jax.numpy as jnp
import numpy as np
from jax.experimental import pallas as pl
from jax.experimental.pallas import tpu as pltpu

N = 1024
E = 32768
C = 2
NUM_HEADS = 4
HD = 32
NUM_LAYERS = 3
OUT_DIM = 4
MAX_SPD = 32


def _norm(x, axis=-1, keepdims=False, eps=1e-8):
    return jnp.sqrt(jnp.sum(x * x, axis=axis, keepdims=keepdims) + eps)


def _layernorm_tuple(s, v, g, b):
    mu = s.mean(-1, keepdims=True)
    var = s.var(-1, keepdims=True)
    s = (s - mu) / jnp.sqrt(var + 1e-5) * g + b
    vn = jnp.sqrt(jnp.mean(jnp.sum(v * v, -1), axis=-1, keepdims=True) + 1e-8)[..., None]
    return s, v / vn


def _gvp(s, v, p, act=None):
    vh = jnp.einsum('...ic,ih->...hc', v, p['Wh'])
    vn = _norm(vh)
    so = jnp.concatenate([s, vn], -1) @ p['Ws'] + p['bs']
    vo = jnp.einsum('...hc,ho->...oc', vh, p['Wv'])
    gate = jax.nn.sigmoid(so @ p['Wg'] + p['bg'])
    vo = vo * gate[..., None]
    if act is not None:
        so = act(so)
    return so, vo


def _gvp_scalar_out(s, v, p):
    vh = jnp.einsum('...ic,ih->...hc', v, p['Wh'])
    return jnp.concatenate([s, _norm(vh)], -1) @ p['Ws'] + p['bs']


def _attn_body(q_ref, k_ref, v_ref, vn_ref, bias_ref, bvc_ref, bvr_ref,
               wo_ref, wvv_ref, outs_ref, outv_ref):
    bvc = bvc_ref[...][:, :1]
    bvr = bvr_ref[...][:1, :]
    bm = jnp.where(bvc == bvr, 0.0, -1e9).astype(jnp.float32)
    scale = 1.0 / np.sqrt(HD)
    for c in range(C):
        am = None
        outs = []
        for h in range(NUM_HEADS):
            qb = q_ref[c, h]
            kb = k_ref[c, h]
            s = jax.lax.dot_general(qb, kb, (((1,), (1,)), ((), ())),
                                    preferred_element_type=jnp.float32)
            s = s * scale + bias_ref[h] + bm
            m = jnp.max(s, axis=-1, keepdims=True)
            e = jnp.exp(s - m)
            a = e / jnp.sum(e, axis=-1, keepdims=True)
            outs.append(jnp.dot(a, v_ref[c, h],
                                preferred_element_type=jnp.float32))
            am = a if am is None else am + a
        o = jnp.concatenate(outs, axis=-1)
        outs_ref[c] = jnp.dot(o, wo_ref[...], preferred_element_type=jnp.float32)
        vm = jnp.dot(am * 0.25, vn_ref[c], preferred_element_type=jnp.float32)
        outv_ref[c] = jnp.dot(vm, wvv_ref[...], preferred_element_type=jnp.float32)


def _fused_attention(l, q, k, v, vn, bias12, bvc, bvr, wo, wvv48, interpret=False):
    bi = 256
    grid = (N // bi,)
    kernel = pl.pallas_call(
        _attn_body,
        grid=grid,
        in_specs=[
            pl.BlockSpec((C, NUM_HEADS, bi, HD), lambda i: (0, 0, i, 0)),
            pl.BlockSpec((C, NUM_HEADS, N, HD), lambda i: (0, 0, 0, 0)),
            pl.BlockSpec((C, NUM_HEADS, N, HD), lambda i: (0, 0, 0, 0)),
            pl.BlockSpec((C, N, 48), lambda i: (0, 0, 0)),
            pl.BlockSpec((NUM_HEADS, bi, N), lambda i: (l, i, 0)),
            pl.BlockSpec((bi, 128), lambda i: (i, 0)),
            pl.BlockSpec((8, N), lambda i: (0, 0)),
            pl.BlockSpec((128, 128), lambda i: (0, 0)),
            pl.BlockSpec((48, 48), lambda i: (0, 0)),
        ],
        out_specs=[
            pl.BlockSpec((C, bi, 128), lambda i: (0, i, 0)),
            pl.BlockSpec((C, bi, 48), lambda i: (0, i, 0)),
        ],
        out_shape=[
            jax.ShapeDtypeStruct((C, N, 128), jnp.float32),
            jax.ShapeDtypeStruct((C, N, 48), jnp.float32),
        ],
        interpret=interpret,
    )
    return kernel(q, k, v, vn, bias12, bvc, bvr, wo, wvv48)


def _encoder_layer(s, v, bias12, l, bvc, bvr, p, interpret=False):
    sn, vn_ = _layernorm_tuple(s, v, p['ln1_g'], p['ln1_b'])
    q = (sn @ p['Wq']).reshape(N, C, NUM_HEADS, HD).transpose(1, 2, 0, 3)
    k = (sn @ p['Wk']).reshape(N, C, NUM_HEADS, HD).transpose(1, 2, 0, 3)
    val = (sn @ p['Wval']).reshape(N, C, NUM_HEADS, HD).transpose(1, 2, 0, 3)
    vnr = vn_.transpose(1, 0, 2, 3).reshape(C, N, 48)
    wvv48 = jnp.kron(p['Wvv'], jnp.eye(3, dtype=jnp.float32))
    outs, outv = _fused_attention(l, q, k, val, vnr, bias12, bvc, bvr,
                                  p['Wo'], wvv48, interpret=interpret)
    s = s + outs.transpose(1, 0, 2)
    v = v + outv.transpose(1, 0, 2).reshape(N, C, 16, 3)
    sn2, vn2 = _layernorm_tuple(s, v, p['ln2_g'], p['ln2_b'])
    fs, fv = _gvp(sn2, vn2, p['ff1'], act=jax.nn.silu)
    fs, fv = _gvp(fs, fv, p['ff2'])
    return s + fs, v + fv


def _dec_msg_body(gdst_ref, gsrc_ref, gecs_ref, eds_ref, edvar_ref,
                  wh_a_ref, wh_m_ref, wh_b_ref,
                  ws_dst_ref, ws_eds_ref, ws_src_ref, ws_vn_ref, bs1_ref,
                  wv1_ref, wg1_ref, bg1_ref,
                  wh2_ref, ws2_s_ref, ws2_vn_ref, bs2_ref,
                  wv2_ref, wg2_ref, bg2_ref,
                  ms_ref, mv_ref):
    f32 = jnp.float32
    dot = functools.partial(jax.lax.dot, precision=None)

    def mm(a, b):
        return jax.lax.dot_general(a, b, (((1,), (0,)), ((), ())),
                                   preferred_element_type=f32)

    arf = edvar_ref[:, 3:4]
    na = 1.0 - arf
    gds = gdst_ref[:, 0:128]
    gss = gsrc_ref[:, 0:128]
    ges = gecs_ref[:, 0:128]
    ssrc = arf * gss + na * ges
    wh_m = wh_m_ref[0:1, :]
    vh = []
    for c in range(3):
        vdst_c = gdst_ref[:, 128 + 16 * c:144 + 16 * c]
        vsrc_c = arf * gsrc_ref[:, 128 + 16 * c:144 + 16 * c] \
            + na * gecs_ref[:, 128 + 16 * c:144 + 16 * c]
        edv_c = edvar_ref[:, c:c + 1]
        vh.append(mm(vdst_c, wh_a_ref[...]) + edv_c * wh_m + mm(vsrc_c, wh_b_ref[...]))
    vn1 = jnp.sqrt(vh[0] * vh[0] + vh[1] * vh[1] + vh[2] * vh[2] + 1e-8)
    so = (mm(gds, ws_dst_ref[...]) + mm(eds_ref[...], ws_eds_ref[...])
          + mm(ssrc, ws_src_ref[...]) + mm(vn1, ws_vn_ref[...]) + bs1_ref[0:1, :])
    gate = jax.nn.sigmoid(mm(so, wg1_ref[...]) + bg1_ref[0:1, :])
    vo = [mm(vh[c], wv1_ref[...]) * gate for c in range(3)]
    so = so * jax.nn.sigmoid(so)
    vh2 = [mm(vo[c], wh2_ref[...]) for c in range(3)]
    vn2 = jnp.sqrt(vh2[0] * vh2[0] + vh2[1] * vh2[1] + vh2[2] * vh2[2] + 1e-8)
    so2 = mm(so, ws2_s_ref[...]) + mm(vn2, ws2_vn_ref[...]) + bs2_ref[0:1, :]
    gate2 = jax.nn.sigmoid(mm(so2, wg2_ref[...]) + bg2_ref[0:1, :])
    ms_ref[...] = so2
    for c in range(3):
        mv_ref[:, 16 * c:16 * (c + 1)] = mm(vh2[c], wv2_ref[...]) * gate2


def _dec_weight_pack(p):
    m1, m2 = p['msg1'], p['msg2']
    wh1 = m1['Wh']
    ws1 = m1['Ws']
    b8 = lambda x: jnp.broadcast_to(x[None, :], (8, x.shape[0]))
    return dict(
        wh_a=wh1[0:16],
        wh_m=jnp.broadcast_to(wh1[16:17], (8, 33)),
        wh_b=wh1[17:33],
        ws_dst=ws1[0:128],
        ws_eds=ws1[128:164],
        ws_src=ws1[164:292],
        ws_vn=ws1[292:325],
        bs1=b8(m1['bs']),
        wv1=m1['Wv'],
        wg1=m1['Wg'],
        bg1=b8(m1['bg']),
        wh2=m2['Wh'],
        ws2_s=m2['Ws'][0:128],
        ws2_vn=m2['Ws'][128:144],
        bs2=b8(m2['bs']),
        wv2=m2['Wv'],
        wg2=m2['Wg'],
        bg2=b8(m2['bg']),
    )


def _dec_messages(gdst, gsrc, gecs, eds, edvar, wp, interpret=False):
    be = 1024
    grid = (E // be,)
    full = lambda shape: pl.BlockSpec(shape, lambda i: tuple(0 for _ in shape))
    row = lambda w: pl.BlockSpec((be, w), lambda i: (i, 0))
    worder = ['wh_a', 'wh_m', 'wh_b', 'ws_dst', 'ws_eds', 'ws_src', 'ws_vn',
              'bs1', 'wv1', 'wg1', 'bg1', 'wh2', 'ws2_s', 'ws2_vn', 'bs2',
              'wv2', 'wg2', 'bg2']
    kernel = pl.pallas_call(
        _dec_msg_body,
        grid=grid,
        in_specs=[row(176), row(176), row(176), row(36), row(8)]
                 + [full(wp[k].shape) for k in worder],
        out_specs=[row(128), row(48)],
        out_shape=[
            jax.ShapeDtypeStruct((E, 128), jnp.float32),
            jax.ShapeDtypeStruct((E, 48), jnp.float32),
        ],
        interpret=interpret,
    )
    return kernel(gdst, gsrc, gecs, eds, edvar, *[wp[k] for k in worder])


def _vc(v):
    return v.transpose(0, 2, 1).reshape(v.shape[0], 48)


def _decoder_layer_fused(hs, hv, src, dst, gecs, eds, edvar, cnt_s, cnt_v, p,
                         interpret=False):
    sn, vn_ = _layernorm_tuple(hs, hv, p['ln1_g'], p['ln1_b'])
    tab = jnp.concatenate([sn, _vc(vn_)], axis=1)
    gdst = tab[dst]
    gsrc = tab[src]
    wp = _dec_weight_pack(p)
    ms, mv = _dec_messages(gdst, gsrc, gecs, eds, edvar, wp, interpret=interpret)
    hs = hs + jax.ops.segment_sum(ms, dst, N) / cnt_s
    mvn = jax.ops.segment_sum(mv, dst, N).reshape(N, 3, 16).transpose(0, 2, 1)
    hv = hv + mvn / cnt_v
    sn2, vn2 = _layernorm_tuple(hs, hv, p['ln2_g'], p['ln2_b'])
    fs, fv = _gvp(sn2, vn2, p['ff1'], act=jax.nn.silu)
    fs, fv = _gvp(fs, fv, p['ff2'])
    return hs + fs, hv + fv


def _decoder_layer(hs, hv, src, dst, ed_s, ed_v, enc_s, enc_v, p):
    n = hs.shape[0]
    sn, vn_ = _layernorm_tuple(hs, hv, p['ln1_g'], p['ln1_b'])
    ar = (src < dst)
    s_src = jnp.where(ar[:, None], sn[src], enc_s[src])
    v_src = jnp.where(ar[:, None, None], vn_[src], enc_v[src])
    ms = jnp.concatenate([sn[dst], ed_s, s_src], -1)
    mv = jnp.concatenate([vn_[dst], ed_v, v_src], -2)
    ms, mv = _gvp(ms, mv, p['msg1'], act=jax.nn.silu)
    ms, mv = _gvp(ms, mv, p['msg2'])
    cnt = jnp.clip(jax.ops.segment_sum(jnp.ones((dst.shape[0],), hs.dtype), dst, n), 1.0, None)
    hs = hs + jax.ops.segment_sum(ms, dst, n) / cnt[:, None]
    hv = hv + jax.ops.segment_sum(mv, dst, n) / cnt[:, None, None]
    sn2, vn2 = _layernorm_tuple(hs, hv, p['ln2_g'], p['ln2_b'])
    fs, fv = _gvp(sn2, vn2, p['ff1'], act=jax.nn.silu)
    fs, fv = _gvp(fs, fv, p['ff2'])
    return hs + fs, hv + fv


def _build_bias12(edge_feat_all, spd_matrix, shortest_path_edges, src, dst, params):
    spd_tab = jnp.concatenate([lp['spd_emb'] for lp in params['enc']], axis=-1)
    path_tab = jnp.concatenate([edge_feat_all @ lp['We_path'] for lp in params['enc']], axis=-1)
    edge_tab = jnp.concatenate([edge_feat_all @ lp['We_bias'] for lp in params['enc']], axis=-1)
    bias = spd_tab[spd_matrix] + path_tab[shortest_path_edges]
    bias = bias.at[dst, src].add(edge_tab)
    return bias.transpose(2, 0, 1)


def _forward(node_s, node_v, edge_s, edge_v, mask_confs, params, edge_index,
             seq, spd_matrix, shortest_path_edges, batch_vec, interpret=False):
    src, dst = edge_index[0], edge_index[1]
    n_conf = jnp.clip(mask_confs.sum(1, keepdims=True), 1.0, None)
    edge_feat_all = (edge_s * mask_confs[src][..., None]).sum(1) / n_conf[src]
    s, v = _layernorm_tuple(node_s, node_v, params['ln_v_g'], params['ln_v_b'])
    s, v = _gvp(s, v, params['W_v'])
    es, ev = _layernorm_tuple(edge_s, edge_v, params['ln_e_g'], params['ln_e_b'])
    es, ev = _gvp(es, ev, params['W_e'])

    bias12 = _build_bias12(edge_feat_all, spd_matrix, shortest_path_edges, src, dst, params)
    bvc = jnp.broadcast_to(batch_vec[:, None], (N, 128)).astype(jnp.int32)
    bvr = jnp.broadcast_to(batch_vec[None, :], (8, N)).astype(jnp.int32)
    for l, lp in enumerate(params['enc']):
        s, v = _encoder_layer(s, v, bias12, l, bvc, bvr, lp, interpret=interpret)

    nclip = jnp.clip(mask_confs.sum(1), 1.0, None)
    s_p = (s * mask_confs[..., None]).sum(1) / nclip[:, None]
    v_p = (v * mask_confs[..., None, None]).sum(1) / nclip[:, None, None]
    mce = mask_confs[src]
    ncl_e = jnp.clip(mce.sum(1), 1.0, None)
    es_p = (es * mce[..., None]).sum(1) / ncl_e[:, None]
    ev_p = (ev * mce[..., None, None]).sum(1) / ncl_e[:, None, None]
    hS = params['W_s'][seq][src]
    hS = jnp.where((src < dst)[:, None], hS, 0.0)
    ed_s = jnp.concatenate([es_p, hS], -1)
    enc_tab = jnp.concatenate([s_p, _vc(v_p)], axis=1)
    gecs = enc_tab[src]
    arf = (src < dst).astype(jnp.float32)
    edvar = jnp.concatenate([ev_p[:, 0, :], arf[:, None],
                             jnp.zeros((E, 4), jnp.float32)], axis=1)
    cnt = jnp.clip(jax.ops.segment_sum(jnp.ones((E,), jnp.float32), dst, N), 1.0, None)
    cnt_s, cnt_v = cnt[:, None], cnt[:, None, None]
    hs, hv = s_p, v_p
    for lp in params['dec']:
        hs, hv = _decoder_layer_fused(hs, hv, src, dst, gecs, ed_s, edvar,
                                      cnt_s, cnt_v, lp, interpret=interpret)
    return _gvp_scalar_out(hs, hv, params['W_out'])


def kernel(node_s, node_v, edge_s, edge_v, edge_index, seq, spd_matrix,
           shortest_path_edges, mask_confs, batch_vec, params):
    return _forward(node_s, node_v, edge_s, edge_v, mask_confs, params,
                    edge_index, seq, spd_matrix, shortest_path_edges, batch_vec)


# speedup vs baseline: 3.2825x; 3.2825x over previous
import functools
import jax
import jax.numpy as jnp
import numpy as np
from jax import lax
from jax.experimental import pallas as pl
from jax.experimental.pallas import tpu as pltpu
from jax.experimental.pallas import tpu_sc as plsc

N = 1024
E = 32768
C = 2
NUM_HEADS = 4
HD = 32
NUM_LAYERS = 3
OUT_DIM = 4
MAX_SPD = 32


def _norm(x, axis=-1, keepdims=False, eps=1e-8):
    return jnp.sqrt(jnp.sum(x * x, axis=axis, keepdims=keepdims) + eps)


def _layernorm_tuple(s, v, g, b):
    mu = s.mean(-1, keepdims=True)
    var = s.var(-1, keepdims=True)
    s = (s - mu) / jnp.sqrt(var + 1e-5) * g + b
    vn = jnp.sqrt(jnp.mean(jnp.sum(v * v, -1), axis=-1, keepdims=True) + 1e-8)[..., None]
    return s, v / vn


def _gvp(s, v, p, act=None):
    vh = jnp.einsum('...ic,ih->...hc', v, p['Wh'])
    vn = _norm(vh)
    so = jnp.concatenate([s, vn], -1) @ p['Ws'] + p['bs']
    vo = jnp.einsum('...hc,ho->...oc', vh, p['Wv'])
    gate = jax.nn.sigmoid(so @ p['Wg'] + p['bg'])
    vo = vo * gate[..., None]
    if act is not None:
        so = act(so)
    return so, vo


def _gvp_scalar_out(s, v, p):
    vh = jnp.einsum('...ic,ih->...hc', v, p['Wh'])
    return jnp.concatenate([s, _norm(vh)], -1) @ p['Ws'] + p['bs']


def _attn_body(q_ref, k_ref, v_ref, vn_ref, bias_ref, bvc_ref, bvr_ref,
               wo_ref, wvv_ref, outs_ref, outv_ref):
    bvc = bvc_ref[...][:, :1]
    bvr = bvr_ref[...][:1, :]
    bm = jnp.where(bvc == bvr, 0.0, -1e9).astype(jnp.float32)
    scale = 1.0 / np.sqrt(HD)
    for c in range(C):
        am = None
        outs = []
        for h in range(NUM_HEADS):
            qb = q_ref[c, h]
            kb = k_ref[c, h]
            s = jax.lax.dot_general(qb, kb, (((1,), (1,)), ((), ())),
                                    preferred_element_type=jnp.float32)
            s = s * scale + bias_ref[h] + bm
            m = jnp.max(s, axis=-1, keepdims=True)
            e = jnp.exp(s - m)
            a = e / jnp.sum(e, axis=-1, keepdims=True)
            outs.append(jnp.dot(a, v_ref[c, h],
                                preferred_element_type=jnp.float32))
            am = a if am is None else am + a
        o = jnp.concatenate(outs, axis=-1)
        outs_ref[c] = jnp.dot(o, wo_ref[...], preferred_element_type=jnp.float32)
        vm = jnp.dot(am * 0.25, vn_ref[c], preferred_element_type=jnp.float32)
        outv_ref[c] = jnp.dot(vm, wvv_ref[...], preferred_element_type=jnp.float32)


def _fused_attention(l, q, k, v, vn, bias12, bvc, bvr, wo, wvv48, interpret=False):
    bi = 256
    grid = (N // bi,)
    kernel = pl.pallas_call(
        _attn_body,
        grid=grid,
        in_specs=[
            pl.BlockSpec((C, NUM_HEADS, bi, HD), lambda i: (0, 0, i, 0)),
            pl.BlockSpec((C, NUM_HEADS, N, HD), lambda i: (0, 0, 0, 0)),
            pl.BlockSpec((C, NUM_HEADS, N, HD), lambda i: (0, 0, 0, 0)),
            pl.BlockSpec((C, N, 48), lambda i: (0, 0, 0)),
            pl.BlockSpec((NUM_HEADS, bi, N), lambda i: (l, i, 0)),
            pl.BlockSpec((bi, 128), lambda i: (i, 0)),
            pl.BlockSpec((8, N), lambda i: (0, 0)),
            pl.BlockSpec((128, 128), lambda i: (0, 0)),
            pl.BlockSpec((48, 48), lambda i: (0, 0)),
        ],
        out_specs=[
            pl.BlockSpec((C, bi, 128), lambda i: (0, i, 0)),
            pl.BlockSpec((C, bi, 48), lambda i: (0, i, 0)),
        ],
        out_shape=[
            jax.ShapeDtypeStruct((C, N, 128), jnp.float32),
            jax.ShapeDtypeStruct((C, N, 48), jnp.float32),
        ],
        interpret=interpret,
    )
    return kernel(q, k, v, vn, bias12, bvc, bvr, wo, wvv48)


def _encoder_layer(s, v, bias12, l, bvc, bvr, p, interpret=False):
    sn, vn_ = _layernorm_tuple(s, v, p['ln1_g'], p['ln1_b'])
    q = (sn @ p['Wq']).reshape(N, C, NUM_HEADS, HD).transpose(1, 2, 0, 3)
    k = (sn @ p['Wk']).reshape(N, C, NUM_HEADS, HD).transpose(1, 2, 0, 3)
    val = (sn @ p['Wval']).reshape(N, C, NUM_HEADS, HD).transpose(1, 2, 0, 3)
    vnr = vn_.transpose(1, 0, 2, 3).reshape(C, N, 48)
    wvv48 = jnp.kron(p['Wvv'], jnp.eye(3, dtype=jnp.float32))
    outs, outv = _fused_attention(l, q, k, val, vnr, bias12, bvc, bvr,
                                  p['Wo'], wvv48, interpret=interpret)
    s = s + outs.transpose(1, 0, 2)
    v = v + outv.transpose(1, 0, 2).reshape(N, C, 16, 3)
    sn2, vn2 = _layernorm_tuple(s, v, p['ln2_g'], p['ln2_b'])
    fs, fv = _gvp(sn2, vn2, p['ff1'], act=jax.nn.silu)
    fs, fv = _gvp(fs, fv, p['ff2'])
    return s + fs, v + fv


def _dec_msg_body(gdst_ref, gsrc_ref, gecs_ref, eds_ref, edvar_ref,
                  wh_a_ref, wh_m_ref, wh_b_ref,
                  ws_dst_ref, ws_eds_ref, ws_src_ref, ws_vn_ref, bs1_ref,
                  wv1_ref, wg1_ref, bg1_ref,
                  wh2_ref, ws2_s_ref, ws2_vn_ref, bs2_ref,
                  wv2_ref, wg2_ref, bg2_ref,
                  ms_ref, mv_ref):
    f32 = jnp.float32
    dot = functools.partial(jax.lax.dot, precision=None)

    def mm(a, b):
        return jax.lax.dot_general(a, b, (((1,), (0,)), ((), ())),
                                   preferred_element_type=f32)

    arf = edvar_ref[:, 3:4]
    na = 1.0 - arf
    gds = gdst_ref[:, 0:128]
    gss = gsrc_ref[:, 0:128]
    ges = gecs_ref[:, 0:128]
    ssrc = arf * gss + na * ges
    wh_m = wh_m_ref[0:1, :]
    vh = []
    for c in range(3):
        vdst_c = gdst_ref[:, 128 + 16 * c:144 + 16 * c]
        vsrc_c = arf * gsrc_ref[:, 128 + 16 * c:144 + 16 * c] \
            + na * gecs_ref[:, 128 + 16 * c:144 + 16 * c]
        edv_c = edvar_ref[:, c:c + 1]
        vh.append(mm(vdst_c, wh_a_ref[...]) + edv_c * wh_m + mm(vsrc_c, wh_b_ref[...]))
    vn1 = jnp.sqrt(vh[0] * vh[0] + vh[1] * vh[1] + vh[2] * vh[2] + 1e-8)
    so = (mm(gds, ws_dst_ref[...]) + mm(eds_ref[...], ws_eds_ref[...])
          + mm(ssrc, ws_src_ref[...]) + mm(vn1, ws_vn_ref[...]) + bs1_ref[0:1, :])
    gate = jax.nn.sigmoid(mm(so, wg1_ref[...]) + bg1_ref[0:1, :])
    vo = [mm(vh[c], wv1_ref[...]) * gate for c in range(3)]
    so = so * jax.nn.sigmoid(so)
    vh2 = [mm(vo[c], wh2_ref[...]) for c in range(3)]
    vn2 = jnp.sqrt(vh2[0] * vh2[0] + vh2[1] * vh2[1] + vh2[2] * vh2[2] + 1e-8)
    so2 = mm(so, ws2_s_ref[...]) + mm(vn2, ws2_vn_ref[...]) + bs2_ref[0:1, :]
    gate2 = jax.nn.sigmoid(mm(so2, wg2_ref[...]) + bg2_ref[0:1, :])
    ms_ref[...] = so2
    for c in range(3):
        mv_ref[:, 16 * c:16 * (c + 1)] = mm(vh2[c], wv2_ref[...]) * gate2


def _dec_weight_pack(p):
    m1, m2 = p['msg1'], p['msg2']
    wh1 = m1['Wh']
    ws1 = m1['Ws']
    b8 = lambda x: jnp.broadcast_to(x[None, :], (8, x.shape[0]))
    return dict(
        wh_a=wh1[0:16],
        wh_m=jnp.broadcast_to(wh1[16:17], (8, 33)),
        wh_b=wh1[17:33],
        ws_dst=ws1[0:128],
        ws_eds=ws1[128:164],
        ws_src=ws1[164:292],
        ws_vn=ws1[292:325],
        bs1=b8(m1['bs']),
        wv1=m1['Wv'],
        wg1=m1['Wg'],
        bg1=b8(m1['bg']),
        wh2=m2['Wh'],
        ws2_s=m2['Ws'][0:128],
        ws2_vn=m2['Ws'][128:144],
        bs2=b8(m2['bs']),
        wv2=m2['Wv'],
        wg2=m2['Wg'],
        bg2=b8(m2['bg']),
    )


def _dec_messages(gdst, gsrc, gecs, eds, edvar, wp, interpret=False):
    be = 1024
    grid = (E // be,)
    full = lambda shape: pl.BlockSpec(shape, lambda i: tuple(0 for _ in shape))
    row = lambda w: pl.BlockSpec((be, w), lambda i: (i, 0))
    worder = ['wh_a', 'wh_m', 'wh_b', 'ws_dst', 'ws_eds', 'ws_src', 'ws_vn',
              'bs1', 'wv1', 'wg1', 'bg1', 'wh2', 'ws2_s', 'ws2_vn', 'bs2',
              'wv2', 'wg2', 'bg2']
    kernel = pl.pallas_call(
        _dec_msg_body,
        grid=grid,
        in_specs=[row(176), row(176), row(176), row(36), row(8)]
                 + [full(wp[k].shape) for k in worder],
        out_specs=[row(128), row(48)],
        out_shape=[
            jax.ShapeDtypeStruct((E, 128), jnp.float32),
            jax.ShapeDtypeStruct((E, 48), jnp.float32),
        ],
        interpret=interpret,
    )
    return kernel(gdst, gsrc, gecs, eds, edvar, *[wp[k] for k in worder])


def _vc(v):
    return v.transpose(0, 2, 1).reshape(v.shape[0], 48)


def _decoder_layer_fused(hs, hv, src, dst, gecs, eds, edvar, cnt_s, cnt_v, p,
                         interpret=False):
    sn, vn_ = _layernorm_tuple(hs, hv, p['ln1_g'], p['ln1_b'])
    tab = jnp.concatenate([sn, _vc(vn_)], axis=1)
    gdst = tab[dst]
    gsrc = tab[src]
    wp = _dec_weight_pack(p)
    ms, mv = _dec_messages(gdst, gsrc, gecs, eds, edvar, wp, interpret=interpret)
    hs = hs + jax.ops.segment_sum(ms, dst, N) / cnt_s
    mvn = jax.ops.segment_sum(mv, dst, N).reshape(N, 3, 16).transpose(0, 2, 1)
    hv = hv + mvn / cnt_v
    sn2, vn2 = _layernorm_tuple(hs, hv, p['ln2_g'], p['ln2_b'])
    fs, fv = _gvp(sn2, vn2, p['ff1'], act=jax.nn.silu)
    fs, fv = _gvp(fs, fv, p['ff2'])
    return hs + fs, hv + fv


def _decoder_layer(hs, hv, src, dst, ed_s, ed_v, enc_s, enc_v, p):
    n = hs.shape[0]
    sn, vn_ = _layernorm_tuple(hs, hv, p['ln1_g'], p['ln1_b'])
    ar = (src < dst)
    s_src = jnp.where(ar[:, None], sn[src], enc_s[src])
    v_src = jnp.where(ar[:, None, None], vn_[src], enc_v[src])
    ms = jnp.concatenate([sn[dst], ed_s, s_src], -1)
    mv = jnp.concatenate([vn_[dst], ed_v, v_src], -2)
    ms, mv = _gvp(ms, mv, p['msg1'], act=jax.nn.silu)
    ms, mv = _gvp(ms, mv, p['msg2'])
    cnt = jnp.clip(jax.ops.segment_sum(jnp.ones((dst.shape[0],), hs.dtype), dst, n), 1.0, None)
    hs = hs + jax.ops.segment_sum(ms, dst, n) / cnt[:, None]
    hv = hv + jax.ops.segment_sum(mv, dst, n) / cnt[:, None, None]
    sn2, vn2 = _layernorm_tuple(hs, hv, p['ln2_g'], p['ln2_b'])
    fs, fv = _gvp(sn2, vn2, p['ff1'], act=jax.nn.silu)
    fs, fv = _gvp(fs, fv, p['ff2'])
    return hs + fs, hv + fv


def _bias_gather_sc(spd_matrix, spe_matrix, spd_tabs, path_tabs):
    mesh = plsc.VectorSubcoreMesh(core_axis_name="c", subcore_axis_name="s")
    rh = 16

    @functools.partial(
        pl.kernel, mesh=mesh,
        out_type=jax.ShapeDtypeStruct((12, N, N), jnp.float32),
        compiler_params=pltpu.CompilerParams(needs_layout_passes=False),
        scratch_types=[
            pltpu.VMEM((rh, N), jnp.int32),
            pltpu.VMEM((rh, N), jnp.int32),
            pltpu.VMEM((rh, N), jnp.float32),
            pltpu.VMEM((128,), jnp.float32),
            pltpu.VMEM((E,), jnp.float32),
        ],
    )
    def k(spd_hbm, spe_hbm, stabs_hbm, ptabs_hbm, out_hbm,
          spd_v, spe_v, outb_v, stab_v, ptab_v):
        wid = lax.axis_index("s") * 2 + lax.axis_index("c")
        for h in range(2):
            base = wid * 32 + h * rh
            pltpu.sync_copy(spd_hbm.at[pl.ds(base, rh)], spd_v)
            pltpu.sync_copy(spe_hbm.at[pl.ds(base, rh)], spe_v)
            for k_ in range(12):
                pltpu.sync_copy(stabs_hbm.at[k_], stab_v)
                pltpu.sync_copy(ptabs_hbm.at[k_], ptab_v)

                def row_body(r, carry):
                    def chunk_body(j, carry2):
                        idx_d = spd_v[r, pl.ds(j * 16, 16)]
                        idx_e = spe_v[r, pl.ds(j * 16, 16)]
                        g = plsc.load_gather(stab_v, [idx_d]) \
                            + plsc.load_gather(ptab_v, [idx_e])
                        outb_v[r, pl.ds(j * 16, 16)] = g
                        return carry2
                    return lax.fori_loop(0, N // 16, chunk_body, carry)

                lax.fori_loop(0, rh, row_body, 0)
                pltpu.sync_copy(outb_v, out_hbm.at[k_, pl.ds(base, rh)])

    return k(spd_matrix, spe_matrix, spd_tabs, path_tabs)


def _build_bias12(edge_feat_all, spd_matrix, shortest_path_edges, src, dst,
                  params, interpret=False):
    spd_tab = jnp.concatenate([lp['spd_emb'] for lp in params['enc']], axis=-1)
    path_tab = jnp.concatenate([edge_feat_all @ lp['We_path'] for lp in params['enc']], axis=-1)
    edge_tab = jnp.concatenate([edge_feat_all @ lp['We_bias'] for lp in params['enc']], axis=-1)
    if interpret:
        bias = spd_tab[spd_matrix] + path_tab[shortest_path_edges]
        bias = bias.transpose(2, 0, 1)
    else:
        spd_tabs = jnp.pad(spd_tab.T, ((0, 0), (0, 128 - MAX_SPD)))
        bias = _bias_gather_sc(spd_matrix.astype(jnp.int32),
                               shortest_path_edges.astype(jnp.int32),
                               spd_tabs, path_tab.T.copy())
    return bias.at[:, dst, src].add(edge_tab.T)


def _forward(node_s, node_v, edge_s, edge_v, mask_confs, params, edge_index,
             seq, spd_matrix, shortest_path_edges, batch_vec, interpret=False):
    src, dst = edge_index[0], edge_index[1]
    n_conf = jnp.clip(mask_confs.sum(1, keepdims=True), 1.0, None)
    edge_feat_all = (edge_s * mask_confs[src][..., None]).sum(1) / n_conf[src]
    s, v = _layernorm_tuple(node_s, node_v, params['ln_v_g'], params['ln_v_b'])
    s, v = _gvp(s, v, params['W_v'])
    es, ev = _layernorm_tuple(edge_s, edge_v, params['ln_e_g'], params['ln_e_b'])
    es, ev = _gvp(es, ev, params['W_e'])

    bias12 = _build_bias12(edge_feat_all, spd_matrix, shortest_path_edges, src,
                           dst, params, interpret=interpret)
    bvc = jnp.broadcast_to(batch_vec[:, None], (N, 128)).astype(jnp.int32)
    bvr = jnp.broadcast_to(batch_vec[None, :], (8, N)).astype(jnp.int32)
    for l, lp in enumerate(params['enc']):
        s, v = _encoder_layer(s, v, bias12, l, bvc, bvr, lp, interpret=interpret)

    nclip = jnp.clip(mask_confs.sum(1), 1.0, None)
    s_p = (s * mask_confs[..., None]).sum(1) / nclip[:, None]
    v_p = (v * mask_confs[..., None, None]).sum(1) / nclip[:, None, None]
    mce = mask_confs[src]
    ncl_e = jnp.clip(mce.sum(1), 1.0, None)
    es_p = (es * mce[..., None]).sum(1) / ncl_e[:, None]
    ev_p = (ev * mce[..., None, None]).sum(1) / ncl_e[:, None, None]
    hS = params['W_s'][seq][src]
    hS = jnp.where((src < dst)[:, None], hS, 0.0)
    ed_s = jnp.concatenate([es_p, hS], -1)
    enc_tab = jnp.concatenate([s_p, _vc(v_p)], axis=1)
    gecs = enc_tab[src]
    arf = (src < dst).astype(jnp.float32)
    edvar = jnp.concatenate([ev_p[:, 0, :], arf[:, None],
                             jnp.zeros((E, 4), jnp.float32)], axis=1)
    cnt = jnp.clip(jax.ops.segment_sum(jnp.ones((E,), jnp.float32), dst, N), 1.0, None)
    cnt_s, cnt_v = cnt[:, None], cnt[:, None, None]
    hs, hv = s_p, v_p
    for lp in params['dec']:
        hs, hv = _decoder_layer_fused(hs, hv, src, dst, gecs, ed_s, edvar,
                                      cnt_s, cnt_v, lp, interpret=interpret)
    return _gvp_scalar_out(hs, hv, params['W_out'])


def kernel(node_s, node_v, edge_s, edge_v, edge_index, seq, spd_matrix,
           shortest_path_edges, mask_confs, batch_vec, params):
    return _forward(node_s, node_v, edge_s, edge_v, mask_confs, params,
                    edge_index, seq, spd_matrix, shortest_path_edges, batch_vec)


# speedup vs baseline: 3.7975x; 1.1569x over previous
import functools
import jax
import jax.numpy as jnp
import numpy as np
from jax import lax
from jax.experimental import pallas as pl
from jax.experimental.pallas import tpu as pltpu
from jax.experimental.pallas import tpu_sc as plsc

N = 1024
E = 32768
C = 2
NUM_HEADS = 4
HD = 32
NUM_LAYERS = 3
OUT_DIM = 4
MAX_SPD = 32


def _norm(x, axis=-1, keepdims=False, eps=1e-8):
    return jnp.sqrt(jnp.sum(x * x, axis=axis, keepdims=keepdims) + eps)


def _layernorm_tuple(s, v, g, b):
    mu = s.mean(-1, keepdims=True)
    var = s.var(-1, keepdims=True)
    s = (s - mu) / jnp.sqrt(var + 1e-5) * g + b
    vn = jnp.sqrt(jnp.mean(jnp.sum(v * v, -1), axis=-1, keepdims=True) + 1e-8)[..., None]
    return s, v / vn


def _gvp(s, v, p, act=None):
    vh = jnp.einsum('...ic,ih->...hc', v, p['Wh'])
    vn = _norm(vh)
    so = jnp.concatenate([s, vn], -1) @ p['Ws'] + p['bs']
    vo = jnp.einsum('...hc,ho->...oc', vh, p['Wv'])
    gate = jax.nn.sigmoid(so @ p['Wg'] + p['bg'])
    vo = vo * gate[..., None]
    if act is not None:
        so = act(so)
    return so, vo


def _gvp_scalar_out(s, v, p):
    vh = jnp.einsum('...ic,ih->...hc', v, p['Wh'])
    return jnp.concatenate([s, _norm(vh)], -1) @ p['Ws'] + p['bs']


def _attn_body(q_ref, k_ref, v_ref, vn_ref, bias_ref, bvc_ref, bvr_ref,
               wo_ref, wvv_ref, outs_ref, outv_ref):
    bvc = bvc_ref[...][:, :1]
    bvr = bvr_ref[...][:1, :]
    bm = jnp.where(bvc == bvr, 0.0, -1e9).astype(jnp.float32)
    scale = 1.0 / np.sqrt(HD)
    for c in range(C):
        am = None
        outs = []
        for h in range(NUM_HEADS):
            qb = q_ref[c, h]
            kb = k_ref[c, h]
            s = jax.lax.dot_general(qb, kb, (((1,), (1,)), ((), ())),
                                    preferred_element_type=jnp.float32)
            s = s * scale + bias_ref[h] + bm
            m = jnp.max(s, axis=-1, keepdims=True)
            e = jnp.exp(s - m)
            a = e / jnp.sum(e, axis=-1, keepdims=True)
            outs.append(jnp.dot(a, v_ref[c, h],
                                preferred_element_type=jnp.float32))
            am = a if am is None else am + a
        o = jnp.concatenate(outs, axis=-1)
        outs_ref[c] = jnp.dot(o, wo_ref[...], preferred_element_type=jnp.float32)
        vm = jnp.dot(am * 0.25, vn_ref[c], preferred_element_type=jnp.float32)
        outv_ref[c] = jnp.dot(vm, wvv_ref[...], preferred_element_type=jnp.float32)


def _fused_attention(l, q, k, v, vn, bias12, bvc, bvr, wo, wvv48, interpret=False):
    bi = 256
    grid = (N // bi,)
    kernel = pl.pallas_call(
        _attn_body,
        grid=grid,
        in_specs=[
            pl.BlockSpec((C, NUM_HEADS, bi, HD), lambda i: (0, 0, i, 0)),
            pl.BlockSpec((C, NUM_HEADS, N, HD), lambda i: (0, 0, 0, 0)),
            pl.BlockSpec((C, NUM_HEADS, N, HD), lambda i: (0, 0, 0, 0)),
            pl.BlockSpec((C, N, 48), lambda i: (0, 0, 0)),
            pl.BlockSpec((NUM_HEADS, bi, N), lambda i: (l, i, 0)),
            pl.BlockSpec((bi, 128), lambda i: (i, 0)),
            pl.BlockSpec((8, N), lambda i: (0, 0)),
            pl.BlockSpec((128, 128), lambda i: (0, 0)),
            pl.BlockSpec((48, 48), lambda i: (0, 0)),
        ],
        out_specs=[
            pl.BlockSpec((C, bi, 128), lambda i: (0, i, 0)),
            pl.BlockSpec((C, bi, 48), lambda i: (0, i, 0)),
        ],
        out_shape=[
            jax.ShapeDtypeStruct((C, N, 128), jnp.float32),
            jax.ShapeDtypeStruct((C, N, 48), jnp.float32),
        ],
        interpret=interpret,
    )
    return kernel(q, k, v, vn, bias12, bvc, bvr, wo, wvv48)


def _encoder_layer(s, v, bias12, l, bvc, bvr, p, interpret=False):
    sn, vn_ = _layernorm_tuple(s, v, p['ln1_g'], p['ln1_b'])
    q = (sn @ p['Wq']).reshape(N, C, NUM_HEADS, HD).transpose(1, 2, 0, 3)
    k = (sn @ p['Wk']).reshape(N, C, NUM_HEADS, HD).transpose(1, 2, 0, 3)
    val = (sn @ p['Wval']).reshape(N, C, NUM_HEADS, HD).transpose(1, 2, 0, 3)
    vnr = vn_.transpose(1, 0, 2, 3).reshape(C, N, 48)
    wvv48 = jnp.kron(p['Wvv'], jnp.eye(3, dtype=jnp.float32))
    outs, outv = _fused_attention(l, q, k, val, vnr, bias12, bvc, bvr,
                                  p['Wo'], wvv48, interpret=interpret)
    s = s + outs.transpose(1, 0, 2)
    v = v + outv.transpose(1, 0, 2).reshape(N, C, 16, 3)
    sn2, vn2 = _layernorm_tuple(s, v, p['ln2_g'], p['ln2_b'])
    fs, fv = _gvp(sn2, vn2, p['ff1'], act=jax.nn.silu)
    fs, fv = _gvp(fs, fv, p['ff2'])
    return s + fs, v + fv


def _dec_msg_body(gdst_ref, gsrc_ref, gecs_ref, eds_ref, edvar_ref,
                  wh_a_ref, wh_m_ref, wh_b_ref,
                  ws_dst_ref, ws_eds_ref, ws_src_ref, ws_vn_ref, bs1_ref,
                  wv1_ref, wg1_ref, bg1_ref,
                  wh2_ref, ws2_s_ref, ws2_vn_ref, bs2_ref,
                  wv2_ref, wg2_ref, bg2_ref,
                  ms_ref, mv_ref):
    f32 = jnp.float32

    def mm(a, b):
        return jax.lax.dot_general(a, b, (((1,), (0,)), ((), ())),
                                   preferred_element_type=f32)

    arf = edvar_ref[:, 3:4]
    na = 1.0 - arf
    gds = gdst_ref[:, 0:128]
    gss = gsrc_ref[:, 0:128]
    ges = gecs_ref[:, 0:128]
    ssrc = arf * gss + na * ges
    wh_m = wh_m_ref[0:1, :]
    vh = []
    for c in range(3):
        vdst_c = gdst_ref[:, 128 + 16 * c:144 + 16 * c]
        vsrc_c = arf * gsrc_ref[:, 128 + 16 * c:144 + 16 * c] \
            + na * gecs_ref[:, 128 + 16 * c:144 + 16 * c]
        edv_c = edvar_ref[:, c:c + 1]
        vh.append(mm(vdst_c, wh_a_ref[...]) + edv_c * wh_m + mm(vsrc_c, wh_b_ref[...]))
    vn1 = jnp.sqrt(vh[0] * vh[0] + vh[1] * vh[1] + vh[2] * vh[2] + 1e-8)
    so = (mm(gds, ws_dst_ref[...]) + mm(eds_ref[...], ws_eds_ref[...])
          + mm(ssrc, ws_src_ref[...]) + mm(vn1, ws_vn_ref[...]) + bs1_ref[0:1, :])
    gate = jax.nn.sigmoid(mm(so, wg1_ref[...]) + bg1_ref[0:1, :])
    vo = [mm(vh[c], wv1_ref[...]) * gate for c in range(3)]
    so = so * jax.nn.sigmoid(so)
    vh2 = [mm(vo[c], wh2_ref[...]) for c in range(3)]
    vn2 = jnp.sqrt(vh2[0] * vh2[0] + vh2[1] * vh2[1] + vh2[2] * vh2[2] + 1e-8)
    so2 = mm(so, ws2_s_ref[...]) + mm(vn2, ws2_vn_ref[...]) + bs2_ref[0:1, :]
    gate2 = jax.nn.sigmoid(mm(so2, wg2_ref[...]) + bg2_ref[0:1, :])
    ms_ref[...] = so2
    for c in range(3):
        mv_ref[:, 16 * c:16 * (c + 1)] = mm(vh2[c], wv2_ref[...]) * gate2


def _sc_gather(tab, idx):
    M = idx.shape[0]
    W = tab.shape[1]
    per_w = M // 32
    chunk = 256
    mesh = plsc.VectorSubcoreMesh(core_axis_name="c", subcore_axis_name="s")

    @functools.partial(
        pl.kernel, mesh=mesh,
        out_type=jax.ShapeDtypeStruct((M, W), jnp.float32),
        compiler_params=pltpu.CompilerParams(needs_layout_passes=False),
        scratch_types=[
            pltpu.VMEM((chunk,), jnp.int32),
            pltpu.VMEM((chunk, W), jnp.float32),
            pltpu.SemaphoreType.DMA,
        ],
    )
    def k(tab_hbm, idx_hbm, out_hbm, idx_v, rows_v, sem):
        wid = lax.axis_index("s") * 2 + lax.axis_index("c")
        base = wid * per_w
        for t in range(per_w // chunk):
            pltpu.sync_copy(idx_hbm.at[pl.ds(base + t * chunk, chunk)], idx_v)
            pltpu.async_copy(tab_hbm.at[idx_v], rows_v, sem).wait()
            pltpu.sync_copy(rows_v, out_hbm.at[pl.ds(base + t * chunk, chunk)])

    return k(tab, idx)


def _dec_weight_pack(p):
    m1, m2 = p['msg1'], p['msg2']
    wh1 = m1['Wh']
    ws1 = m1['Ws']
    b8 = lambda x: jnp.broadcast_to(x[None, :], (8, x.shape[0]))
    return dict(
        wh_a=wh1[0:16],
        wh_m=jnp.broadcast_to(wh1[16:17], (8, 33)),
        wh_b=wh1[17:33],
        ws_dst=ws1[0:128],
        ws_eds=ws1[128:164],
        ws_src=ws1[164:292],
        ws_vn=ws1[292:325],
        bs1=b8(m1['bs']),
        wv1=m1['Wv'],
        wg1=m1['Wg'],
        bg1=b8(m1['bg']),
        wh2=m2['Wh'],
        ws2_s=m2['Ws'][0:128],
        ws2_vn=m2['Ws'][128:144],
        bs2=b8(m2['bs']),
        wv2=m2['Wv'],
        wg2=m2['Wg'],
        bg2=b8(m2['bg']),
    )


def _dec_messages(gdst, gsrc, gecs, eds, edvar, wp, interpret=False):
    be = 1024
    grid = (E // be,)
    full = lambda shape: pl.BlockSpec(shape, lambda i: tuple(0 for _ in shape))
    row = lambda w: pl.BlockSpec((be, w), lambda i: (i, 0))
    worder = ['wh_a', 'wh_m', 'wh_b', 'ws_dst', 'ws_eds', 'ws_src', 'ws_vn',
              'bs1', 'wv1', 'wg1', 'bg1', 'wh2', 'ws2_s', 'ws2_vn', 'bs2',
              'wv2', 'wg2', 'bg2']
    nblk = E // be
    kernel = pl.pallas_call(
        _dec_msg_body,
        grid=grid,
        in_specs=[pl.BlockSpec((be, 256), lambda i: (i, 0)),
                  pl.BlockSpec((be, 256), lambda i: (i + nblk, 0)),
                  row(256), row(36), row(8)]
                 + [full(wp[k].shape) for k in worder],
        out_specs=[row(128), row(48)],
        out_shape=[
            jax.ShapeDtypeStruct((E, 128), jnp.float32),
            jax.ShapeDtypeStruct((E, 48), jnp.float32),
        ],
        interpret=interpret,
    )
    return kernel(gdst, gsrc, gecs, eds, edvar, *[wp[k] for k in worder])


def _vc(v):
    return v.transpose(0, 2, 1).reshape(v.shape[0], 48)


def _decoder_layer_fused(hs, hv, src, dst, gecs, eds, edvar, cnt_s, cnt_v, p,
                         interpret=False):
    sn, vn_ = _layernorm_tuple(hs, hv, p['ln1_g'], p['ln1_b'])
    tab = jnp.concatenate([sn, _vc(vn_), jnp.zeros((N, 80), jnp.float32)],
                          axis=1)
    if interpret:
        gall = jnp.concatenate([tab[dst], tab[src]], axis=0)
    else:
        gall = _sc_gather(tab, jnp.concatenate([dst, src]).astype(jnp.int32))
    wp = _dec_weight_pack(p)
    ms, mv = _dec_messages(gall, gall, gecs, eds, edvar, wp, interpret=interpret)
    hs = hs + jax.ops.segment_sum(ms, dst, N) / cnt_s
    mvn = jax.ops.segment_sum(mv, dst, N).reshape(N, 3, 16).transpose(0, 2, 1)
    hv = hv + mvn / cnt_v
    sn2, vn2 = _layernorm_tuple(hs, hv, p['ln2_g'], p['ln2_b'])
    fs, fv = _gvp(sn2, vn2, p['ff1'], act=jax.nn.silu)
    fs, fv = _gvp(fs, fv, p['ff2'])
    return hs + fs, hv + fv


def _decoder_layer(hs, hv, src, dst, ed_s, ed_v, enc_s, enc_v, p):
    n = hs.shape[0]
    sn, vn_ = _layernorm_tuple(hs, hv, p['ln1_g'], p['ln1_b'])
    ar = (src < dst)
    s_src = jnp.where(ar[:, None], sn[src], enc_s[src])
    v_src = jnp.where(ar[:, None, None], vn_[src], enc_v[src])
    ms = jnp.concatenate([sn[dst], ed_s, s_src], -1)
    mv = jnp.concatenate([vn_[dst], ed_v, v_src], -2)
    ms, mv = _gvp(ms, mv, p['msg1'], act=jax.nn.silu)
    ms, mv = _gvp(ms, mv, p['msg2'])
    cnt = jnp.clip(jax.ops.segment_sum(jnp.ones((dst.shape[0],), hs.dtype), dst, n), 1.0, None)
    hs = hs + jax.ops.segment_sum(ms, dst, n) / cnt[:, None]
    hv = hv + jax.ops.segment_sum(mv, dst, n) / cnt[:, None, None]
    sn2, vn2 = _layernorm_tuple(hs, hv, p['ln2_g'], p['ln2_b'])
    fs, fv = _gvp(sn2, vn2, p['ff1'], act=jax.nn.silu)
    fs, fv = _gvp(fs, fv, p['ff2'])
    return hs + fs, hv + fv


def _bias_gather_sc(spd_matrix, spe_matrix, spd_tabs, path_tabs):
    mesh = plsc.VectorSubcoreMesh(core_axis_name="c", subcore_axis_name="s")
    rh = 16

    @functools.partial(
        pl.kernel, mesh=mesh,
        out_type=jax.ShapeDtypeStruct((12, N, N), jnp.float32),
        compiler_params=pltpu.CompilerParams(needs_layout_passes=False),
        scratch_types=[
            pltpu.VMEM((rh, N), jnp.int32),
            pltpu.VMEM((rh, N), jnp.int32),
            pltpu.VMEM((rh, N), jnp.float32),
            pltpu.VMEM((128,), jnp.float32),
            pltpu.VMEM((E,), jnp.float32),
        ],
    )
    def k(spd_hbm, spe_hbm, stabs_hbm, ptabs_hbm, out_hbm,
          spd_v, spe_v, outb_v, stab_v, ptab_v):
        wid = lax.axis_index("s") * 2 + lax.axis_index("c")
        for h in range(2):
            base = wid * 32 + h * rh
            pltpu.sync_copy(spd_hbm.at[pl.ds(base, rh)], spd_v)
            pltpu.sync_copy(spe_hbm.at[pl.ds(base, rh)], spe_v)
            for k_ in range(12):
                pltpu.sync_copy(stabs_hbm.at[k_], stab_v)
                pltpu.sync_copy(ptabs_hbm.at[k_], ptab_v)

                def row_body(r, carry):
                    def chunk_body(j, carry2):
                        idx_d = spd_v[r, pl.ds(j * 16, 16)]
                        idx_e = spe_v[r, pl.ds(j * 16, 16)]
                        g = plsc.load_gather(stab_v, [idx_d]) \
                            + plsc.load_gather(ptab_v, [idx_e])
                        outb_v[r, pl.ds(j * 16, 16)] = g
                        return carry2
                    return lax.fori_loop(0, N // 16, chunk_body, carry)

                lax.fori_loop(0, rh, row_body, 0)
                pltpu.sync_copy(outb_v, out_hbm.at[k_, pl.ds(base, rh)])

    return k(spd_matrix, spe_matrix, spd_tabs, path_tabs)


def _build_bias12(edge_feat_all, spd_matrix, shortest_path_edges, src, dst,
                  params, interpret=False):
    spd_tab = jnp.concatenate([lp['spd_emb'] for lp in params['enc']], axis=-1)
    path_tab = jnp.concatenate([edge_feat_all @ lp['We_path'] for lp in params['enc']], axis=-1)
    edge_tab = jnp.concatenate([edge_feat_all @ lp['We_bias'] for lp in params['enc']], axis=-1)
    if interpret:
        bias = spd_tab[spd_matrix] + path_tab[shortest_path_edges]
        bias = bias.transpose(2, 0, 1)
    else:
        spd_tabs = jnp.pad(spd_tab.T, ((0, 0), (0, 128 - MAX_SPD)))
        bias = _bias_gather_sc(spd_matrix.astype(jnp.int32),
                               shortest_path_edges.astype(jnp.int32),
                               spd_tabs, path_tab.T.copy())
    return bias.at[:, dst, src].add(edge_tab.T)


def _forward(node_s, node_v, edge_s, edge_v, mask_confs, params, edge_index,
             seq, spd_matrix, shortest_path_edges, batch_vec, interpret=False):
    src, dst = edge_index[0], edge_index[1]
    n_conf = jnp.clip(mask_confs.sum(1, keepdims=True), 1.0, None)
    edge_feat_all = (edge_s * mask_confs[src][..., None]).sum(1) / n_conf[src]
    s, v = _layernorm_tuple(node_s, node_v, params['ln_v_g'], params['ln_v_b'])
    s, v = _gvp(s, v, params['W_v'])
    es, ev = _layernorm_tuple(edge_s, edge_v, params['ln_e_g'], params['ln_e_b'])
    es, ev = _gvp(es, ev, params['W_e'])

    bias12 = _build_bias12(edge_feat_all, spd_matrix, shortest_path_edges, src,
                           dst, params, interpret=interpret)
    bvc = jnp.broadcast_to(batch_vec[:, None], (N, 128)).astype(jnp.int32)
    bvr = jnp.broadcast_to(batch_vec[None, :], (8, N)).astype(jnp.int32)
    for l, lp in enumerate(params['enc']):
        s, v = _encoder_layer(s, v, bias12, l, bvc, bvr, lp, interpret=interpret)

    nclip = jnp.clip(mask_confs.sum(1), 1.0, None)
    s_p = (s * mask_confs[..., None]).sum(1) / nclip[:, None]
    v_p = (v * mask_confs[..., None, None]).sum(1) / nclip[:, None, None]
    mce = mask_confs[src]
    ncl_e = jnp.clip(mce.sum(1), 1.0, None)
    es_p = (es * mce[..., None]).sum(1) / ncl_e[:, None]
    ev_p = (ev * mce[..., None, None]).sum(1) / ncl_e[:, None, None]
    hS = params['W_s'][seq][src]
    hS = jnp.where((src < dst)[:, None], hS, 0.0)
    ed_s = jnp.concatenate([es_p, hS], -1)
    enc_tab = jnp.concatenate([s_p, _vc(v_p), jnp.zeros((N, 80), jnp.float32)],
                              axis=1)
    if interpret:
        gecs = enc_tab[src]
    else:
        gecs = _sc_gather(enc_tab, src.astype(jnp.int32))
    arf = (src < dst).astype(jnp.float32)
    edvar = jnp.concatenate([ev_p[:, 0, :], arf[:, None],
                             jnp.zeros((E, 4), jnp.float32)], axis=1)
    cnt = jnp.clip(jax.ops.segment_sum(jnp.ones((E,), jnp.float32), dst, N), 1.0, None)
    cnt_s, cnt_v = cnt[:, None], cnt[:, None, None]
    hs, hv = s_p, v_p
    for lp in params['dec']:
        hs, hv = _decoder_layer_fused(hs, hv, src, dst, gecs, ed_s, edvar,
                                      cnt_s, cnt_v, lp, interpret=interpret)
    return _gvp_scalar_out(hs, hv, params['W_out'])


def kernel(node_s, node_v, edge_s, edge_v, edge_index, seq, spd_matrix,
           shortest_path_edges, mask_confs, batch_vec, params):
    return _forward(node_s, node_v, edge_s, edge_v, mask_confs, params,
                    edge_index, seq, spd_matrix, shortest_path_edges, batch_vec)
